# baseline (speedup 1.0000x reference)
import jax
import jax.numpy as jnp
from jax import lax
from jax.experimental import pallas as pl
from jax.experimental.pallas import tpu as pltpu
from jax.experimental.pallas import tpu_sc as plsc

_EMBED = 64
_CTX = 200
_NC = 2
_NS = 16
_NW = _NC * _NS
_CHUNK = 200
_PAD = 224
_LANES = 16
_EG = _EMBED // _LANES


def _emb_body(tb_hbm, e2_hbm, p_hbm, out_hbm, p_v, *scratch):
    idx_v = scratch[0:2]
    idx2_v = scratch[2:4]
    rows2_v = scratch[4:6]
    out_v = scratch[6:8]
    isem = scratch[8:10]
    gsem = scratch[10:12]
    osem = scratch[12:14]

    wid = lax.axis_index("s") * _NC + lax.axis_index("c")
    n_per_w = tb_hbm.shape[0] // _NW
    steps = n_per_w // _CHUNK
    base = wid * n_per_w

    pltpu.sync_copy(p_hbm, p_v)

    def issue_idx(g, b):
        pltpu.async_copy(tb_hbm.at[pl.ds(base + g * _CHUNK, _CHUNK)],
                         idx_v[b].at[pl.ds(0, _CHUNK)], isem[b])

    def wait_idx(b):
        pltpu.make_async_copy(tb_hbm.at[pl.ds(0, _CHUNK)],
                              idx_v[b].at[pl.ds(0, _CHUNK)], isem[b]).wait()

    def halve_idx(b):
        for s in range(_PAD // _LANES):
            sl = pl.ds(s * _LANES, _LANES)
            idx2_v[b][sl] = idx_v[b][sl] >> 1

    def fire_gather(b):
        pltpu.async_copy(
            e2_hbm.at[idx2_v[b].at[pl.ds(0, _CHUNK)]],
            rows2_v[b],
            gsem[b],
        )

    def drain_gather(b):
        pltpu.make_async_copy(e2_hbm.at[pl.ds(0, _CHUNK)],
                              rows2_v[b], gsem[b]).wait()

    def issue_store(g, b):
        pltpu.async_copy(out_v[b],
                         out_hbm.at[pl.ds(base + g * _CHUNK, _CHUNK)],
                         osem[b])

    def wait_store(b):
        pltpu.make_async_copy(out_v[b],
                              out_hbm.at[pl.ds(0, _CHUNK)], osem[b]).wait()

    def select_add(b):
        def row(r, c):
            t = idx_v[b][pl.ds(r, _LANES)][0]
            off = (t & 1) << 6
            for j in range(_EG):
                dst = pl.ds(j * _LANES, _LANES)
                src = pl.ds(off + j * _LANES, _LANES)
                out_v[b][r, dst] = rows2_v[b][r, src] + p_v[r, dst]
            return c

        lax.fori_loop(0, _CHUNK, row, 0)

    issue_idx(0, 0)
    issue_idx(1, 1)
    wait_idx(0)
    halve_idx(0)
    fire_gather(0)

    def body(g, carry):
        b = lax.rem(g, 2)

        def buf(fn):
            pl.when(b == 0)(lambda: fn(0))
            pl.when(b == 1)(lambda: fn(1))

        def _next(b1):
            wait_idx(b1)
            halve_idx(b1)
            fire_gather(b1)
        pl.when(g + 1 < steps)(lambda: buf(lambda b0: _next(1 - b0)))

        def _cur(b0):
            drain_gather(b0)
            pl.when(g >= 2)(lambda: wait_store(b0))
            select_add(b0)
        buf(_cur)

        def _post(b0):
            issue_store(g, b0)
            pl.when(g + 2 < steps)(lambda: issue_idx(g + 2, b0))
        buf(_post)
        return carry

    lax.fori_loop(0, steps, body, 0)

    for b in range(2):
        wait_store(b)


def kernel(token_batch, E, P):
    batch, seq = token_batch.shape
    n = batch * seq
    tb = token_batch.reshape(n).astype(jnp.int32)
    e2 = E.reshape(E.shape[0] // 2, 2 * _EMBED)

    emb = pl.kernel(
        _emb_body,
        out_type=jax.ShapeDtypeStruct((n, _EMBED), jnp.float32),
        mesh=plsc.VectorSubcoreMesh(core_axis_name="c", subcore_axis_name="s"),
        scratch_types=(
            [pltpu.VMEM((_CTX, _EMBED), jnp.float32)]
            + [pltpu.VMEM((_PAD,), jnp.int32) for _ in range(2)]
            + [pltpu.VMEM((_PAD,), jnp.int32) for _ in range(2)]
            + [pltpu.VMEM((_CHUNK, 2 * _EMBED), jnp.float32) for _ in range(2)]
            + [pltpu.VMEM((_CHUNK, _EMBED), jnp.float32) for _ in range(2)]
            + [pltpu.SemaphoreType.DMA for _ in range(6)]
        ),
        compiler_params=pltpu.CompilerParams(use_tc_tiling_on_sc=True),
    )
    out = emb(tb, e2, P)
    return out.reshape(batch, seq, _EMBED)


# speedup vs baseline: 1.3716x; 1.3716x over previous
import jax
import jax.numpy as jnp
from jax import lax
from jax.experimental import pallas as pl
from jax.experimental.pallas import tpu as pltpu
from jax.experimental.pallas import tpu_sc as plsc

_EMBED = 64
_CTX = 200
_NC = 2
_NS = 16
_NW = _NC * _NS
_CHUNK = 400
_SUB = 80
_NSUB = _CHUNK // _SUB
_NBUF = 4
_LANES = 16
_EG = _EMBED // _LANES
_K = _CHUNK // _CTX


def _emb_body(tb_hbm, e_hbm, p_hbm, out_hbm, p_v, *scratch):
    idx_v = scratch[0:_NBUF]
    rows_v = scratch[_NBUF:2 * _NBUF]
    isem = scratch[2 * _NBUF:3 * _NBUF]
    gsem = scratch[3 * _NBUF:4 * _NBUF]
    osem = scratch[4 * _NBUF:5 * _NBUF]

    wid = lax.axis_index("s") * _NC + lax.axis_index("c")
    n_per_w = tb_hbm.shape[0] // _NW
    steps = n_per_w // _CHUNK
    base = wid * n_per_w

    pltpu.sync_copy(p_hbm, p_v)

    def issue_idx(g, b):
        pltpu.async_copy(tb_hbm.at[pl.ds(base + g * _CHUNK, _CHUNK)],
                         idx_v[b], isem[b])

    def wait_idx(b):
        pltpu.make_async_copy(tb_hbm.at[pl.ds(0, _CHUNK)],
                              idx_v[b], isem[b]).wait()

    def fire_gather(b):
        for s in range(_NSUB):
            pltpu.async_copy(
                e_hbm.at[idx_v[b].at[pl.ds(s * _SUB, _SUB)]],
                rows_v[b].at[pl.ds(s * _SUB, _SUB)],
                gsem[b],
            )

    def drain_gather(b):
        pltpu.make_async_copy(e_hbm.at[pl.ds(0, _CHUNK)],
                              rows_v[b], gsem[b]).wait()

    seq_base = wid * (n_per_w // _CTX)

    def issue_store(g, b):
        for k in range(_K):
            pltpu.async_copy(rows_v[b].at[pl.ds(k * _CTX, _CTX)],
                             out_hbm.at[seq_base + g * _K + k],
                             osem[b])

    def wait_store(b):
        for k in range(_K):
            pltpu.make_async_copy(rows_v[b].at[pl.ds(k * _CTX, _CTX)],
                                  out_hbm.at[0], osem[b]).wait()

    for b in range(_NBUF):
        issue_idx(b, b)
    wait_idx(0)
    fire_gather(0)

    def outer(i, carry):
        g0 = i * _NBUF
        for b in range(_NBUF):
            g = g0 + b
            b1 = (b + 1) % _NBUF
            drain_gather(b)
            pl.when(g + _NBUF < steps)(lambda: issue_idx(g + _NBUF, b))
            pl.when(g >= _NBUF - 1)(lambda: wait_store(b1))

            def _fire():
                wait_idx(b1)
                fire_gather(b1)
            pl.when(g + 1 < steps)(_fire)

            def add_row(p, c):
                for j in range(_EG):
                    pv = p_v[p, pl.ds(j * _LANES, _LANES)]
                    for k in range(_K):
                        r = p + k * _CTX
                        rows_v[b][r, pl.ds(j * _LANES, _LANES)] = (
                            rows_v[b][r, pl.ds(j * _LANES, _LANES)] + pv
                        )
                return c

            lax.fori_loop(0, _CTX, add_row, 0)
            issue_store(g, b)
        return carry

    lax.fori_loop(0, steps // _NBUF, outer, 0)

    for b in range(1, _NBUF):
        wait_store(b)


def kernel(token_batch, E, P):
    batch, seq = token_batch.shape
    n = batch * seq
    tb = token_batch.reshape(n).astype(jnp.int32)

    emb = pl.kernel(
        _emb_body,
        out_type=jax.ShapeDtypeStruct((batch, seq, _EMBED), jnp.float32),
        mesh=plsc.VectorSubcoreMesh(core_axis_name="c", subcore_axis_name="s"),
        scratch_types=(
            [pltpu.VMEM((_CTX, _EMBED), jnp.float32)]
            + [pltpu.VMEM((_CHUNK,), jnp.int32) for _ in range(_NBUF)]
            + [pltpu.VMEM((_CHUNK, _EMBED), jnp.float32) for _ in range(_NBUF)]
            + [pltpu.SemaphoreType.DMA for _ in range(3 * _NBUF)]
        ),
        compiler_params=pltpu.CompilerParams(use_tc_tiling_on_sc=False),
    )
    return emb(tb, E, P)


# speedup vs baseline: 1.3740x; 1.0017x over previous
import jax
import jax.numpy as jnp
from jax import lax
from jax.experimental import pallas as pl
from jax.experimental.pallas import tpu as pltpu
from jax.experimental.pallas import tpu_sc as plsc

_EMBED = 64
_CTX = 200
_NC = 2
_NS = 16
_NW = _NC * _NS
_CHUNK = 400
_SUB = 80
_NSUB = _CHUNK // _SUB
_NBUF = 4
_LANES = 16
_EG = _EMBED // _LANES
_K = _CHUNK // _CTX


def _emb_body(tb_hbm, e_hbm, p_hbm, out_hbm, p_v, *scratch):
    idx_v = scratch[0:_NBUF]
    rows_v = scratch[_NBUF:2 * _NBUF]
    isem = scratch[2 * _NBUF:3 * _NBUF]
    gsem = scratch[3 * _NBUF:4 * _NBUF]
    osem = scratch[4 * _NBUF:5 * _NBUF]

    wid = lax.axis_index("s") * _NC + lax.axis_index("c")
    n_per_w = (tb_hbm.shape[0] * tb_hbm.shape[1]) // _NW
    steps = n_per_w // _CHUNK
    sb = wid * (n_per_w // _CTX)

    pltpu.sync_copy(p_hbm, p_v)

    def issue_idx(g, b):
        for k in range(_K):
            pltpu.async_copy(tb_hbm.at[sb + g * _K + k],
                             idx_v[b].at[pl.ds(k * _CTX, _CTX)], isem[b])

    def wait_idx(b):
        for k in range(_K):
            pltpu.make_async_copy(tb_hbm.at[0],
                                  idx_v[b].at[pl.ds(k * _CTX, _CTX)],
                                  isem[b]).wait()

    def fire_gather(b):
        for s in range(_NSUB):
            pltpu.async_copy(
                e_hbm.at[idx_v[b].at[pl.ds(s * _SUB, _SUB)]],
                rows_v[b].at[pl.ds(s * _SUB, _SUB)],
                gsem[b],
            )

    def drain_gather(b):
        pltpu.make_async_copy(e_hbm.at[pl.ds(0, _CHUNK)],
                              rows_v[b], gsem[b]).wait()

    seq_base = wid * (n_per_w // _CTX)

    def issue_store(g, b):
        for k in range(_K):
            pltpu.async_copy(rows_v[b].at[pl.ds(k * _CTX, _CTX)],
                             out_hbm.at[seq_base + g * _K + k],
                             osem[b])

    def wait_store(b):
        for k in range(_K):
            pltpu.make_async_copy(rows_v[b].at[pl.ds(k * _CTX, _CTX)],
                                  out_hbm.at[0], osem[b]).wait()

    for b in range(_NBUF):
        issue_idx(b, b)
    wait_idx(0)
    fire_gather(0)

    def outer(i, carry):
        g0 = i * _NBUF
        for b in range(_NBUF):
            g = g0 + b
            b1 = (b + 1) % _NBUF
            drain_gather(b)
            pl.when(g + _NBUF < steps)(lambda: issue_idx(g + _NBUF, b))
            pl.when(g >= _NBUF - 1)(lambda: wait_store(b1))

            def _fire():
                wait_idx(b1)
                fire_gather(b1)
            pl.when(g + 1 < steps)(_fire)

            def add_row(p, c):
                for j in range(_EG):
                    pv = p_v[p, pl.ds(j * _LANES, _LANES)]
                    for k in range(_K):
                        r = p + k * _CTX
                        rows_v[b][r, pl.ds(j * _LANES, _LANES)] = (
                            rows_v[b][r, pl.ds(j * _LANES, _LANES)] + pv
                        )
                return c

            lax.fori_loop(0, _CTX, add_row, 0)
            issue_store(g, b)
        return carry

    lax.fori_loop(0, steps // _NBUF, outer, 0)

    for b in range(1, _NBUF):
        wait_store(b)


def kernel(token_batch, E, P):
    batch, seq = token_batch.shape
    tb = token_batch.astype(jnp.int32)

    emb = pl.kernel(
        _emb_body,
        out_type=jax.ShapeDtypeStruct((batch, seq, _EMBED), jnp.float32),
        mesh=plsc.VectorSubcoreMesh(core_axis_name="c", subcore_axis_name="s"),
        scratch_types=(
            [pltpu.VMEM((_CTX, _EMBED), jnp.float32)]
            + [pltpu.VMEM((_CHUNK,), jnp.int32) for _ in range(_NBUF)]
            + [pltpu.VMEM((_CHUNK, _EMBED), jnp.float32) for _ in range(_NBUF)]
            + [pltpu.SemaphoreType.DMA for _ in range(3 * _NBUF)]
        ),
        compiler_params=pltpu.CompilerParams(use_tc_tiling_on_sc=False),
    )
    return emb(tb, E, P)
